# baseline (speedup 1.0000x reference)
import functools

import jax
import jax.numpy as jnp
from jax.experimental import pallas as pl
from jax.experimental.pallas import tpu as pltpu


def _fused_kernel(x_ref, w1_ref, b1_ref, w2_ref, b2_ref, e_ref, o_ref, *, hw):
    x = x_ref[...]
    C = x.shape[0]
    s = (jnp.sum(x, axis=(1, 2)) * (1.0 / hw)
         + jnp.max(x, axis=(1, 2))).reshape(C, 1)
    h = jnp.dot(w1_ref[...], s, preferred_element_type=jnp.float32) + b1_ref[...]
    h = jnp.maximum(h, 0.0)
    logits = jnp.dot(w2_ref[...], h, preferred_element_type=jnp.float32) + b2_ref[...]
    m = jnp.max(logits, axis=0, keepdims=True)
    p = jnp.exp(logits - m)
    a = p / jnp.sum(p, axis=0, keepdims=True)
    scale = jnp.dot(e_ref[...], a, preferred_element_type=jnp.float32)
    o_ref[...] = (x * scale[:, :, None]).astype(o_ref.dtype)


def kernel(x, w1, b1, gamma, beta, run_mean, run_var, w2, b2):
    eps = 1e-5
    B, C, H, W = x.shape
    inter = w1.shape[0]
    groups = w2.shape[0]
    cpg = C // groups
    hw = H * W

    g = gamma / jnp.sqrt(run_var + eps)
    w1e = (w1 * g[:, None]).astype(jnp.float32)
    b1e = (g * (b1 - run_mean) + beta).reshape(inter, 1).astype(jnp.float32)
    w2f = w2.astype(jnp.float32)
    b2c = b2.reshape(groups, 1).astype(jnp.float32)
    E = (jnp.arange(C)[:, None] // cpg == jnp.arange(groups)[None, :]).astype(jnp.float32)

    fused = functools.partial(_fused_kernel, hw=hw)
    out = pl.pallas_call(
        fused,
        out_shape=jax.ShapeDtypeStruct((B, C, H, W), x.dtype),
        grid=(B,),
        in_specs=[
            pl.BlockSpec((pl.Squeezed(), C, H, W), lambda b: (b, 0, 0, 0)),
            pl.BlockSpec((inter, C), lambda b: (0, 0)),
            pl.BlockSpec((inter, 1), lambda b: (0, 0)),
            pl.BlockSpec((groups, inter), lambda b: (0, 0)),
            pl.BlockSpec((groups, 1), lambda b: (0, 0)),
            pl.BlockSpec((C, groups), lambda b: (0, 0)),
        ],
        out_specs=pl.BlockSpec((pl.Squeezed(), C, H, W), lambda b: (b, 0, 0, 0)),
        compiler_params=pltpu.CompilerParams(
            dimension_semantics=("parallel",),
            vmem_limit_bytes=48 * 1024 * 1024),
    )(x, w1e, b1e, w2f, b2c, E)
    return out


# speedup vs baseline: 2.3088x; 2.3088x over previous
import functools

import jax
import jax.numpy as jnp
from jax.experimental import pallas as pl
from jax.experimental.pallas import tpu as pltpu


def _fused_kernel(x_ref, w1_ref, b1_ref, w2_ref, b2_ref, e_ref, o_ref, *, hw):
    x = x_ref[...]
    s = (jnp.sum(x, axis=1, keepdims=True) * (1.0 / hw)
         + jnp.max(x, axis=1, keepdims=True))
    h = jnp.dot(w1_ref[...], s, preferred_element_type=jnp.float32) + b1_ref[...]
    h = jnp.maximum(h, 0.0)
    logits = jnp.dot(w2_ref[...], h, preferred_element_type=jnp.float32) + b2_ref[...]
    m = jnp.max(logits, axis=0, keepdims=True)
    p = jnp.exp(logits - m)
    a = p / jnp.sum(p, axis=0, keepdims=True)
    scale = jnp.dot(e_ref[...], a, preferred_element_type=jnp.float32)
    o_ref[...] = (x * scale).astype(o_ref.dtype)


def kernel(x, w1, b1, gamma, beta, run_mean, run_var, w2, b2):
    eps = 1e-5
    B, C, H, W = x.shape
    inter = w1.shape[0]
    groups = w2.shape[0]
    cpg = C // groups
    hw = H * W

    g = gamma / jnp.sqrt(run_var + eps)
    w1e = (w1 * g[:, None]).astype(jnp.float32)
    b1e = (g * (b1 - run_mean) + beta).reshape(inter, 1).astype(jnp.float32)
    w2f = w2.astype(jnp.float32)
    b2c = b2.reshape(groups, 1).astype(jnp.float32)
    E = (jnp.arange(C)[:, None] // cpg == jnp.arange(groups)[None, :]).astype(jnp.float32)

    x3 = x.reshape(B, C, hw)
    fused = functools.partial(_fused_kernel, hw=hw)
    out = pl.pallas_call(
        fused,
        out_shape=jax.ShapeDtypeStruct((B, C, hw), x.dtype),
        grid=(B,),
        in_specs=[
            pl.BlockSpec((pl.Squeezed(), C, hw), lambda b: (b, 0, 0)),
            pl.BlockSpec((inter, C), lambda b: (0, 0)),
            pl.BlockSpec((inter, 1), lambda b: (0, 0)),
            pl.BlockSpec((groups, inter), lambda b: (0, 0)),
            pl.BlockSpec((groups, 1), lambda b: (0, 0)),
            pl.BlockSpec((C, groups), lambda b: (0, 0)),
        ],
        out_specs=pl.BlockSpec((pl.Squeezed(), C, hw), lambda b: (b, 0, 0)),
        compiler_params=pltpu.CompilerParams(
            dimension_semantics=("parallel",),
            vmem_limit_bytes=48 * 1024 * 1024),
    )(x3, w1e, b1e, w2f, b2c, E)
    return out.reshape(B, C, H, W)
